# Initial kernel scaffold; baseline (speedup 1.0000x reference)
import jax
import jax.numpy as jnp
from jax.experimental import pallas as pl


def kernel(box_regression, cls_logits, anchors, target_boxes, target_labels):
    raise NotImplementedError("write your pallas kernel here")


# speedup vs baseline: 1.2479x; 1.2479x over previous
import jax
import jax.numpy as jnp
from jax.experimental import pallas as pl
from jax.experimental.pallas import tpu as pltpu

_VAR0, _VAR1 = 0.1, 0.2
_TOPK = 50
_T1 = 0.5
_ALPHA = 0.5
_SL1_W = 0.75
_SL1_B = 0.11

_A = 30000
_AP = 30080
_G = 16
_CH = 6016
_NCH = 5


def _huber(v):
    av = jnp.abs(v)
    return jnp.where(av < _SL1_B, 0.5 / _SL1_B * v * v, av - 0.5 * _SL1_B)


def _focal(x):
    return x * x * (-jnp.clip(jnp.log(1.0 - x), -100.0, None))


def _body(cls_ref, brT_ref, aT_ref, tb_ref, ohGC_ref, lead_ref, same_ref,
          out_ref, mq_s, iou_s, mask_s):
    tx0 = tb_ref[0, :, 0:1]
    ty0 = tb_ref[0, :, 1:2]
    tx1 = tb_ref[0, :, 2:3]
    ty1 = tb_ref[0, :, 3:4]
    area_t = (tx1 - tx0) * (ty1 - ty0)
    ohm = ohGC_ref[0]

    def iou_vs(bx0, by0, bx1, by1):
        iw = jnp.clip(jnp.minimum(tx1, bx1) - jnp.maximum(tx0, bx0), 0.0, None)
        ih = jnp.clip(jnp.minimum(ty1, by1) - jnp.maximum(ty0, by0), 0.0, None)
        inter = iw * ih
        area_b = (bx1 - bx0) * (by1 - by0)
        return inter / (area_t + area_b - inter)

    def slices(ci):
        base = pl.multiple_of(ci * _CH, 128)
        acx = aT_ref[0:1, pl.ds(base, _CH)]
        acy = aT_ref[1:2, pl.ds(base, _CH)]
        aw = aT_ref[2:3, pl.ds(base, _CH)]
        ah = aT_ref[3:4, pl.ds(base, _CH)]
        bcx = brT_ref[0, 0:1, pl.ds(base, _CH)]
        bcy = brT_ref[0, 1:2, pl.ds(base, _CH)]
        bw = brT_ref[0, 2:3, pl.ds(base, _CH)]
        bh = brT_ref[0, 3:4, pl.ds(base, _CH)]
        return base, acx, acy, aw, ah, bcx, bcy, bw, bh

    lane0 = jax.lax.broadcasted_iota(jnp.int32, (1, _CH), 1)

    def pass1(ci, carry):
        maxiou, fsum = carry
        base, acx, acy, aw, ah, bcx, bcy, bw, bh = slices(ci)
        valid = lane0 + base * jnp.int32(1) < _A
        mq = iou_vs(acx - aw * 0.5, acy - ah * 0.5,
                    acx + aw * 0.5, acy + ah * 0.5)
        mq_s[:, pl.ds(base, _CH)] = jnp.where(valid, mq, -1.0)
        dcx = acx + bcx * _VAR0 * aw
        dcy = acy + bcy * _VAR0 * ah
        dw = aw * jnp.exp(bw * _VAR1)
        dh = ah * jnp.exp(bh * _VAR1)
        iou2 = iou_vs(dcx - dw * 0.5, dcy - dh * 0.5,
                      dcx + dw * 0.5, dcy + dh * 0.5)
        iou2 = jnp.where(valid, iou2, 0.0)
        iou_s[:, pl.ds(base, _CH)] = iou2
        maxiou = jnp.maximum(maxiou, jnp.max(iou2, axis=1, keepdims=True))
        p = jax.nn.sigmoid(cls_ref[0, pl.ds(base, _CH), :])
        fsum = fsum + jnp.sum(_focal(p))
        return maxiou, fsum

    maxiou, fsum = jax.lax.fori_loop(
        0, _NCH, pass1,
        (jnp.full((_G, 1), 0.0, jnp.float32), jnp.float32(0.0)))
    t2 = jnp.clip(maxiou, _T1 + 1e-12, None)

    mask_s[...] = jnp.zeros((_G, _AP), jnp.float32)
    sub = jax.lax.broadcasted_iota(jnp.int32, (_G, _AP), 1)

    def topk_it(_, carry):
        s = mq_s[...]
        m = jnp.max(s, axis=1, keepdims=True)
        idx = jnp.min(jnp.where(s == m, sub, _AP), axis=1, keepdims=True)
        sel = sub == idx
        mask_s[...] = jnp.where(sel, 1.0, mask_s[...])
        mq_s[...] = jnp.where(sel, -3.0, s)
        return carry

    jax.lax.fori_loop(0, _TOPK, topk_it, 0)

    def pass2(ci, carry):
        bag_n, bag_d, csum = carry
        base, acx, acy, aw, ah, bcx, bcy, bw, bh = slices(ci)
        mcp = jax.nn.sigmoid(
            jax.lax.dot_general(ohm, cls_ref[0, pl.ds(base, _CH), :],
                                (((1,), (1,)), ((), ())),
                                precision=jax.lax.Precision.HIGHEST,
                                preferred_element_type=jnp.float32))
        ecx = ((tx0 + tx1) * 0.5 - acx) / (_VAR0 * aw)
        ecy = ((ty0 + ty1) * 0.5 - acy) / (_VAR0 * ah)
        ew = jnp.log((tx1 - tx0) / aw) / _VAR1
        eh = jnp.log((ty1 - ty0) / ah) / _VAR1
        reg = _SL1_W * (_huber(ecx - bcx) + _huber(ecy - bcy)
                        + _huber(ew - bw) + _huber(eh - bh))
        logit = mcp * jnp.exp(-reg)
        den = 1.0 / jnp.clip(1.0 - logit, 1e-12, None)
        num = logit * den
        msk = mask_s[:, pl.ds(base, _CH)] > 0.5
        bag_n = bag_n + jnp.sum(jnp.where(msk, num, 0.0), axis=1, keepdims=True)
        bag_d = bag_d + jnp.sum(jnp.where(msk, den, 0.0), axis=1, keepdims=True)
        obp = jnp.clip((iou_s[:, pl.ds(base, _CH)] - _T1) / (t2 - _T1),
                       0.0, 1.0)
        gmax = jnp.zeros((_G, _CH), jnp.float32)
        for g in range(_G):
            gmax = jnp.maximum(gmax, same_ref[0, :, g:g + 1] * obp[g:g + 1, :])
        lead = lead_ref[0, :, 0:1]
        corr = lead * (_focal(mcp * (1.0 - gmax)) - _focal(mcp))
        csum = csum + jnp.sum(corr)
        return bag_n, bag_d, csum

    zero16 = jnp.zeros((_G, 1), jnp.float32)
    bag_n, bag_d, csum = jax.lax.fori_loop(
        0, _NCH, pass2, (zero16, zero16, jnp.float32(0.0)))

    bag = bag_n / bag_d
    pos = jnp.sum(-jnp.clip(jnp.log(bag), -100.0, None))
    neg = fsum + csum
    iov = jax.lax.broadcasted_iota(jnp.int32, (1, 1, 128), 2)
    out_ref[...] = jnp.where(iov == 0, pos, jnp.where(iov == 1, neg, 0.0))


def kernel(box_regression, cls_logits, anchors, target_boxes, target_labels):
    B, A, C = cls_logits.shape
    G = target_boxes.shape[1]
    pad = _AP - A
    cls_p = jnp.pad(cls_logits, ((0, 0), (0, pad), (0, 0)),
                    constant_values=-100.0)
    br_p = jnp.pad(box_regression, ((0, 0), (0, pad), (0, 0)))
    a_pad_rows = jnp.tile(jnp.array([[0.0, 0.0, 1.0, 1.0]], jnp.float32),
                          (pad, 1))
    anchors_p = jnp.concatenate([anchors, a_pad_rows], axis=0)
    brT = jnp.swapaxes(br_p, 1, 2)
    aT = jnp.swapaxes(anchors_p, 0, 1)
    lb = target_labels.astype(jnp.int32)
    oh = (lb[..., None] == jnp.arange(C, dtype=jnp.int32)[None, None, :]
          ).astype(jnp.float32)
    same = (lb[:, :, None] == lb[:, None, :]).astype(jnp.float32)
    first = jnp.argmax(same, axis=2)
    lead = (first == jnp.arange(G)[None, :]).astype(jnp.float32)
    lead3 = jnp.tile(lead[:, :, None], (1, 1, 128))

    out = pl.pallas_call(
        _body,
        grid=(B,),
        in_specs=[
            pl.BlockSpec((1, _AP, C), lambda b: (b, 0, 0)),
            pl.BlockSpec((1, 4, _AP), lambda b: (b, 0, 0)),
            pl.BlockSpec((4, _AP), lambda b: (0, 0)),
            pl.BlockSpec((1, G, 4), lambda b: (b, 0, 0)),
            pl.BlockSpec((1, G, C), lambda b: (b, 0, 0)),
            pl.BlockSpec((1, G, 128), lambda b: (b, 0, 0)),
            pl.BlockSpec((1, G, G), lambda b: (b, 0, 0)),
        ],
        out_specs=pl.BlockSpec((1, 1, 128), lambda b: (b, 0, 0)),
        out_shape=jax.ShapeDtypeStruct((B, 1, 128), jnp.float32),
        scratch_shapes=[
            pltpu.VMEM((_G, _AP), jnp.float32),
            pltpu.VMEM((_G, _AP), jnp.float32),
            pltpu.VMEM((_G, _AP), jnp.float32),
        ],
    )(cls_p, brT, aT, target_boxes, oh, lead3, same)

    pos = jnp.sum(out[:, 0, 0]) / (B * G) * _ALPHA
    neg = jnp.sum(out[:, 0, 1]) / (B * G * _TOPK) * (1.0 - _ALPHA)
    return pos, neg


# speedup vs baseline: 1.7361x; 1.3912x over previous
import jax
import jax.numpy as jnp
from jax.experimental import pallas as pl
from jax.experimental.pallas import tpu as pltpu

_VAR0, _VAR1 = 0.1, 0.2
_TOPK = 50
_T1 = 0.5
_ALPHA = 0.5
_SL1_W = 0.75
_SL1_B = 0.11

_A = 30000
_AP = 30080
_G = 16
_CH = 6016
_NCH = 5


def _huber(v):
    av = jnp.abs(v)
    return jnp.where(av < _SL1_B, 0.5 / _SL1_B * v * v, av - 0.5 * _SL1_B)


def _focal(x):
    return x * x * (-jnp.clip(jnp.log(1.0 - x), -100.0, None))


def _body(cls_ref, brT_ref, aT_ref, tb_ref, ohGC_ref, lead_ref, same_ref,
          out_ref, mq_s, iou_s):
    tx0 = tb_ref[0, :, 0:1]
    ty0 = tb_ref[0, :, 1:2]
    tx1 = tb_ref[0, :, 2:3]
    ty1 = tb_ref[0, :, 3:4]
    area_t = (tx1 - tx0) * (ty1 - ty0)
    ohm = ohGC_ref[0]

    def iou_vs(bx0, by0, bx1, by1):
        iw = jnp.clip(jnp.minimum(tx1, bx1) - jnp.maximum(tx0, bx0), 0.0, None)
        ih = jnp.clip(jnp.minimum(ty1, by1) - jnp.maximum(ty0, by0), 0.0, None)
        inter = iw * ih
        area_b = (bx1 - bx0) * (by1 - by0)
        return inter / (area_t + area_b - inter)

    def slices(ci):
        base = pl.multiple_of(ci * _CH, 128)
        acx = aT_ref[0:1, pl.ds(base, _CH)]
        acy = aT_ref[1:2, pl.ds(base, _CH)]
        aw = aT_ref[2:3, pl.ds(base, _CH)]
        ah = aT_ref[3:4, pl.ds(base, _CH)]
        bcx = brT_ref[0, 0:1, pl.ds(base, _CH)]
        bcy = brT_ref[0, 1:2, pl.ds(base, _CH)]
        bw = brT_ref[0, 2:3, pl.ds(base, _CH)]
        bh = brT_ref[0, 3:4, pl.ds(base, _CH)]
        return base, acx, acy, aw, ah, bcx, bcy, bw, bh

    lane0 = jax.lax.broadcasted_iota(jnp.int32, (1, _CH), 1)

    def pass1(ci, carry):
        maxiou, fsum = carry
        base, acx, acy, aw, ah, bcx, bcy, bw, bh = slices(ci)
        valid = lane0 + base * jnp.int32(1) < _A
        mq = iou_vs(acx - aw * 0.5, acy - ah * 0.5,
                    acx + aw * 0.5, acy + ah * 0.5)
        mq_s[:, pl.ds(base, _CH)] = jnp.where(valid, mq, -1.0)
        dcx = acx + bcx * _VAR0 * aw
        dcy = acy + bcy * _VAR0 * ah
        dw = aw * jnp.exp(bw * _VAR1)
        dh = ah * jnp.exp(bh * _VAR1)
        iou2 = iou_vs(dcx - dw * 0.5, dcy - dh * 0.5,
                      dcx + dw * 0.5, dcy + dh * 0.5)
        iou2 = jnp.where(valid, iou2, 0.0)
        iou_s[:, pl.ds(base, _CH)] = iou2
        maxiou = jnp.maximum(maxiou, jnp.max(iou2, axis=1, keepdims=True))
        p = jax.nn.sigmoid(cls_ref[0, pl.ds(base, _CH), :])
        fsum = fsum + jnp.sum(_focal(p))
        return maxiou, fsum

    maxiou, fsum = jax.lax.fori_loop(
        0, _NCH, pass1,
        (jnp.full((_G, 1), 0.0, jnp.float32), jnp.float32(0.0)))
    t2 = jnp.clip(maxiou, _T1 + 1e-12, None)

    def count_gt(t):
        return jnp.sum(jnp.where(mq_s[...] > t, 1.0, 0.0), axis=1,
                       keepdims=True)

    def bs_it(_, carry):
        lo, hi = carry
        mid = jax.lax.div(lo + hi + 1, jnp.int32(2))
        t = jax.lax.bitcast_convert_type(mid, jnp.float32)
        small = count_gt(t) <= jnp.float32(_TOPK - 1)
        lo = jnp.where(small, lo, mid)
        hi = jnp.where(small, mid, hi)
        return lo, hi

    lo0 = jnp.full((_G, 1), -1, jnp.int32)
    hi0 = jnp.full((_G, 1), 0x3F800000, jnp.int32)
    _, hi = jax.lax.fori_loop(0, 31, bs_it, (lo0, hi0))
    v50 = jax.lax.bitcast_convert_type(hi, jnp.float32)
    need = jnp.float32(_TOPK) - count_gt(v50)

    def pass2(ci, carry):
        bag_n, bag_d, csum, tie_base = carry
        base, acx, acy, aw, ah, bcx, bcy, bw, bh = slices(ci)
        mcp = jax.nn.sigmoid(
            jax.lax.dot_general(ohm, cls_ref[0, pl.ds(base, _CH), :],
                                (((1,), (1,)), ((), ())),
                                precision=jax.lax.Precision.HIGHEST,
                                preferred_element_type=jnp.float32))
        ecx = ((tx0 + tx1) * 0.5 - acx) / (_VAR0 * aw)
        ecy = ((ty0 + ty1) * 0.5 - acy) / (_VAR0 * ah)
        ew = jnp.log((tx1 - tx0) / aw) / _VAR1
        eh = jnp.log((ty1 - ty0) / ah) / _VAR1
        reg = _SL1_W * (_huber(ecx - bcx) + _huber(ecy - bcy)
                        + _huber(ew - bw) + _huber(eh - bh))
        logit = mcp * jnp.exp(-reg)
        den = 1.0 / jnp.clip(1.0 - logit, 1e-12, None)
        num = logit * den
        mqc = mq_s[:, pl.ds(base, _CH)]
        tie = jnp.where(mqc == v50, 1.0, 0.0)
        tcs = tie
        n = 1
        while n < _CH:
            tcs = tcs + jnp.concatenate(
                [jnp.zeros((_G, n), jnp.float32), tcs[:, :_CH - n]], axis=1)
            n *= 2
        msk = (mqc > v50) | ((tie > 0.5) & (tie_base + tcs - tie < need))
        tie_base = tie_base + tcs[:, _CH - 1:_CH]
        bag_n = bag_n + jnp.sum(jnp.where(msk, num, 0.0), axis=1, keepdims=True)
        bag_d = bag_d + jnp.sum(jnp.where(msk, den, 0.0), axis=1, keepdims=True)
        obp = jnp.clip((iou_s[:, pl.ds(base, _CH)] - _T1) / (t2 - _T1),
                       0.0, 1.0)
        gmax = jnp.zeros((_G, _CH), jnp.float32)
        for g in range(_G):
            gmax = jnp.maximum(gmax, same_ref[0, :, g:g + 1] * obp[g:g + 1, :])
        lead = lead_ref[0, :, 0:1]
        corr = lead * (_focal(mcp * (1.0 - gmax)) - _focal(mcp))
        csum = csum + jnp.sum(corr)
        return bag_n, bag_d, csum, tie_base

    zero16 = jnp.zeros((_G, 1), jnp.float32)
    bag_n, bag_d, csum, _ = jax.lax.fori_loop(
        0, _NCH, pass2, (zero16, zero16, jnp.float32(0.0), zero16))

    bag = bag_n / bag_d
    pos = jnp.sum(-jnp.clip(jnp.log(bag), -100.0, None))
    neg = fsum + csum
    iov = jax.lax.broadcasted_iota(jnp.int32, (1, 1, 128), 2)
    out_ref[...] = jnp.where(iov == 0, pos, jnp.where(iov == 1, neg, 0.0))


def kernel(box_regression, cls_logits, anchors, target_boxes, target_labels):
    B, A, C = cls_logits.shape
    G = target_boxes.shape[1]
    pad = _AP - A
    cls_p = jnp.pad(cls_logits, ((0, 0), (0, pad), (0, 0)),
                    constant_values=-100.0)
    br_p = jnp.pad(box_regression, ((0, 0), (0, pad), (0, 0)))
    a_pad_rows = jnp.tile(jnp.array([[0.0, 0.0, 1.0, 1.0]], jnp.float32),
                          (pad, 1))
    anchors_p = jnp.concatenate([anchors, a_pad_rows], axis=0)
    brT = jnp.swapaxes(br_p, 1, 2)
    aT = jnp.swapaxes(anchors_p, 0, 1)
    lb = target_labels.astype(jnp.int32)
    oh = (lb[..., None] == jnp.arange(C, dtype=jnp.int32)[None, None, :]
          ).astype(jnp.float32)
    same = (lb[:, :, None] == lb[:, None, :]).astype(jnp.float32)
    first = jnp.argmax(same, axis=2)
    lead = (first == jnp.arange(G)[None, :]).astype(jnp.float32)
    lead3 = jnp.tile(lead[:, :, None], (1, 1, 128))

    out = pl.pallas_call(
        _body,
        grid=(B,),
        in_specs=[
            pl.BlockSpec((1, _AP, C), lambda b: (b, 0, 0)),
            pl.BlockSpec((1, 4, _AP), lambda b: (b, 0, 0)),
            pl.BlockSpec((4, _AP), lambda b: (0, 0)),
            pl.BlockSpec((1, G, 4), lambda b: (b, 0, 0)),
            pl.BlockSpec((1, G, C), lambda b: (b, 0, 0)),
            pl.BlockSpec((1, G, 128), lambda b: (b, 0, 0)),
            pl.BlockSpec((1, G, G), lambda b: (b, 0, 0)),
        ],
        out_specs=pl.BlockSpec((1, 1, 128), lambda b: (b, 0, 0)),
        out_shape=jax.ShapeDtypeStruct((B, 1, 128), jnp.float32),
        scratch_shapes=[
            pltpu.VMEM((_G, _AP), jnp.float32),
            pltpu.VMEM((_G, _AP), jnp.float32),
        ],
    )(cls_p, brT, aT, target_boxes, oh, lead3, same)

    pos = jnp.sum(out[:, 0, 0]) / (B * G) * _ALPHA
    neg = jnp.sum(out[:, 0, 1]) / (B * G * _TOPK) * (1.0 - _ALPHA)
    return pos, neg
